# baseline (speedup 1.0000x reference)
import functools

import jax
import jax.numpy as jnp
from jax import lax
from jax.experimental import pallas as pl
from jax.experimental.pallas import tpu as pltpu
from jax.experimental.pallas import tpu_sc as plsc

E = 8
V = 100000
D = 32
B = 4096

NC = 2
NS = 16
LANES = 16

VPAD = 100096
BLK = 5120
NBLK = 20
_BLKW = [BLK] * 19 + [2688]
TW = 128
TSPLIT = (V - TW) - 19 * BLK
NIV = B // LANES

_mesh = plsc.VectorSubcoreMesh(core_axis_name="c", subcore_axis_name="s")


@functools.partial(
    pl.kernel,
    mesh=_mesh,
    out_type=jax.ShapeDtypeStruct((E, D, B), jnp.float32),
    scratch_types=[
        pltpu.VMEM((B,), jnp.int32),
        pltpu.VMEM((B + NBLK * LANES,), jnp.int32),
        pltpu.VMEM((LANES,), jnp.int32),
        pltpu.VMEM((64,), jnp.int32),
        pltpu.VMEM((64,), jnp.int32),
        pltpu.VMEM((2, 8, BLK), jnp.float32),
        pltpu.VMEM((8, TW), jnp.float32),
        pltpu.VMEM((8, B), jnp.float32),
        pltpu.SemaphoreType.DMA,
        pltpu.SemaphoreType.DMA,
    ],
    compiler_params=pltpu.CompilerParams(
        use_tc_tiling_on_sc=True,
        disable_bounds_checks=True,
        needs_layout_passes=False,
    ),
)
def _gather(idx_hbm, wt_hbm, tail_hbm, out_hbm, idx_v, hv, tmp_v, cnt_v,
            offs_v, bufs, tail_v, out_v, sem_a, sem_b):
    wid = lax.axis_index("s") * NC + lax.axis_index("c")
    e = wid // 4
    r8 = pl.multiple_of((wid % 4) * 8, 8)
    lanes = lax.iota(jnp.int32, LANES)

    def runs(sk):
        tmp_v[...] = sk
        prv = plsc.load_gather(tmp_v, [jnp.maximum(lanes - 1, 0)])
        nxt = plsc.load_gather(tmp_v, [jnp.minimum(lanes + 1, LANES - 1)])
        newrun = (lanes == 0) | (sk != prv)
        lastrun = (lanes == LANES - 1) | (sk != nxt)
        rank = lanes - plsc.cummax(jnp.where(newrun, lanes, 0))
        return rank, lastrun

    def bucket_of(v):
        return lax.shift_right_logical(
            lax.shift_right_logical(v, 10) * 13108, 16
        )

    def fire(j):
        w = _BLKW[j]
        return pltpu.async_copy(
            wt_hbm.at[e, pl.ds(r8, 8), pl.ds(j * BLK, w)],
            bufs.at[j % 2, :, pl.ds(0, w)],
            sem_a if j % 2 == 0 else sem_b,
        )

    cps = {0: fire(0), 1: fire(1)}
    pltpu.sync_copy(idx_hbm.at[pl.ds(pl.multiple_of(e * B, B), B)], idx_v)
    pltpu.sync_copy(tail_hbm.at[e, pl.ds(r8, 8), :], tail_v)

    zeros = jnp.zeros((LANES,), jnp.int32)
    for g in range(4):
        cnt_v[pl.ds(g * LANES, LANES)] = zeros

    ones = jnp.full((LANES,), 1, jnp.int32)

    def cnt_body(i2, carry):
        for u in range(2):
            v = idx_v[pl.ds((i2 * 2 + u) * LANES, LANES)]
            plsc.addupdate_scatter(cnt_v, [bucket_of(v)], ones)
        return carry

    lax.fori_loop(0, NIV // 2, cnt_body, 0)

    tot = jnp.int32(0)
    for g in range(4):
        cg = cnt_v[pl.ds(g * LANES, LANES)]
        cgp = lax.bitwise_and(cg + (LANES - 1), -LANES)
        inc = plsc.cumsum(cgp)
        offs_v[pl.ds(g * LANES, LANES)] = inc - cgp + tot
        tot = tot + jnp.sum(cgp)

    def scalar_at(ref, j):
        x = ref[pl.ds((j // LANES) * LANES, LANES)]
        return jnp.sum(jnp.where(lanes == j % LANES, x, 0))

    def place_body(i2, carry):
        for u in range(2):
            i = i2 * 2 + u
            v = idx_v[pl.ds(i * LANES, LANES)]
            packed = v * B + (lanes + i * LANES)
            sk, sval = plsc.sort_key_val(bucket_of(v), packed)
            rank, lastrun = runs(sk)
            base = plsc.load_gather(offs_v, [sk])
            plsc.store_scatter(hv, [base + rank], sval)
            plsc.addupdate_scatter(offs_v, [sk], rank + 1, mask=lastrun)
        return carry

    lax.fori_loop(0, NIV // 2, place_body, 0)

    for j in range(NBLK):
        cps[j].wait()
        buf = bufs.at[j % 2]
        cj = scalar_at(cnt_v, j)
        n0 = scalar_at(offs_v, j) - cj

        def hit_body(k, carry):
            hval = hv[pl.ds(n0 + k * LANES, LANES)]
            valid = lanes + k * LANES < cj
            b = lax.bitwise_and(hval, B - 1)
            col = lax.shift_right_logical(hval, 12) - j * BLK
            if j < NBLK - 1:
                for r in range(8):
                    rfull = jnp.full((LANES,), r, jnp.int32)
                    vals = plsc.load_gather(buf, [rfull, col], mask=valid)
                    plsc.store_scatter(out_v, [rfull, b], vals, mask=valid)
            else:
                vs = valid & (col < TSPLIT)
                vt = valid & (col >= TSPLIT)
                tcol = col + 19 * BLK - (V - TW)
                for r in range(8):
                    rfull = jnp.full((LANES,), r, jnp.int32)
                    vals = plsc.load_gather(buf, [rfull, col], mask=vs)
                    plsc.store_scatter(out_v, [rfull, b], vals, mask=vs)
                    tvals = plsc.load_gather(tail_v, [rfull, tcol], mask=vt)
                    plsc.store_scatter(out_v, [rfull, b], tvals, mask=vt)
            return carry

        lax.fori_loop(0, (cj + LANES - 1) // LANES, hit_body, 0)
        if j + 2 < NBLK:
            cps[j + 2] = fire(j + 2)
    pltpu.sync_copy(out_v, out_hbm.at[e, pl.ds(r8, 8), :])


def kernel(indices, weight):
    wt = weight.transpose(0, 2, 1)
    tail = weight[:, V - TW:, :].transpose(0, 2, 1)
    out = _gather(indices.astype(jnp.int32).reshape(-1), wt, tail)
    return out.transpose(0, 2, 1)


# speedup vs baseline: 1.0343x; 1.0343x over previous
import functools

import jax
import jax.numpy as jnp
from jax import lax
from jax.experimental import pallas as pl
from jax.experimental.pallas import tpu as pltpu
from jax.experimental.pallas import tpu_sc as plsc

E = 8
V = 100000
D = 32
B = 4096

NC = 2
NS = 16
LANES = 16

VPAD = 100096
BLK = 5120
NBLK = 20
_BLKW = [BLK] * 19 + [2688]
TW = 128
TSPLIT = (V - TW) - 19 * BLK
NIV = B // LANES

_mesh = plsc.VectorSubcoreMesh(core_axis_name="c", subcore_axis_name="s")


@functools.partial(
    pl.kernel,
    mesh=_mesh,
    out_type=jax.ShapeDtypeStruct((E, D, B), jnp.float32),
    scratch_types=[
        pltpu.VMEM((B,), jnp.int32),
        pltpu.VMEM((B + NBLK * LANES,), jnp.int32),
        pltpu.VMEM((LANES,), jnp.int32),
        pltpu.VMEM((64,), jnp.int32),
        pltpu.VMEM((64,), jnp.int32),
        pltpu.VMEM((2, 8, BLK), jnp.float32),
        pltpu.VMEM((8, TW), jnp.float32),
        pltpu.VMEM((8, B), jnp.float32),
        pltpu.SemaphoreType.DMA,
        pltpu.SemaphoreType.DMA,
    ],
    compiler_params=pltpu.CompilerParams(
        use_tc_tiling_on_sc=True,
        disable_bounds_checks=True,
        needs_layout_passes=False,
    ),
)
def _gather(idx_hbm, wt_hbm, tail_hbm, out_hbm, idx_v, hv, tmp_v, cnt_v,
            offs_v, bufs, tail_v, out_v, sem_a, sem_b):
    wid = lax.axis_index("s") * NC + lax.axis_index("c")
    e = wid // 4
    r8 = pl.multiple_of((wid % 4) * 8, 8)
    lanes = lax.iota(jnp.int32, LANES)
    pltpu.sync_copy(idx_hbm.at[pl.ds(pl.multiple_of(e * B, B), B)], idx_v)
    pltpu.sync_copy(tail_hbm.at[e, pl.ds(r8, 8), :], tail_v)

    def runs(sk):
        tmp_v[...] = sk
        prv = plsc.load_gather(tmp_v, [jnp.maximum(lanes - 1, 0)])
        nxt = plsc.load_gather(tmp_v, [jnp.minimum(lanes + 1, LANES - 1)])
        newrun = (lanes == 0) | (sk != prv)
        lastrun = (lanes == LANES - 1) | (sk != nxt)
        rank = lanes - plsc.cummax(jnp.where(newrun, lanes, 0))
        return rank, lastrun

    def bucket_of(v):
        return lax.shift_right_logical(
            lax.shift_right_logical(v, 10) * 13108, 16
        )

    def fire(j):
        w = _BLKW[j]
        return pltpu.async_copy(
            wt_hbm.at[e, pl.ds(r8, 8), pl.ds(j * BLK, w)],
            bufs.at[j % 2, :, pl.ds(0, w)],
            sem_a if j % 2 == 0 else sem_b,
        )

    cps = {0: fire(0), 1: fire(1)}

    zeros = jnp.zeros((LANES,), jnp.int32)
    for g in range(4):
        cnt_v[pl.ds(g * LANES, LANES)] = zeros

    ones = jnp.full((LANES,), 1, jnp.int32)

    def cnt_body(i2, carry):
        for u in range(2):
            v = idx_v[pl.ds((i2 * 2 + u) * LANES, LANES)]
            plsc.addupdate_scatter(cnt_v, [bucket_of(v)], ones)
        return carry

    lax.fori_loop(0, NIV // 2, cnt_body, 0)

    tot = jnp.int32(0)
    for g in range(4):
        cg = cnt_v[pl.ds(g * LANES, LANES)]
        cgp = lax.bitwise_and(cg + (LANES - 1), -LANES)
        inc = plsc.cumsum(cgp)
        offs_v[pl.ds(g * LANES, LANES)] = inc - cgp + tot
        tot = tot + jnp.sum(cgp)

    def scalar_at(ref, j):
        x = ref[pl.ds((j // LANES) * LANES, LANES)]
        return jnp.sum(jnp.where(lanes == j % LANES, x, 0))

    def place_body(i2, carry):
        for u in range(2):
            i = i2 * 2 + u
            v = idx_v[pl.ds(i * LANES, LANES)]
            packed = v * B + (lanes + i * LANES)
            sk, sval = plsc.sort_key_val(bucket_of(v), packed)
            rank, lastrun = runs(sk)
            base = plsc.load_gather(offs_v, [sk])
            plsc.store_scatter(hv, [base + rank], sval)
            plsc.addupdate_scatter(offs_v, [sk], rank + 1, mask=lastrun)
        return carry

    lax.fori_loop(0, NIV // 2, place_body, 0)

    for j in range(NBLK):
        cps[j].wait()
        buf = bufs.at[j % 2]
        cj = scalar_at(cnt_v, j)
        n0 = scalar_at(offs_v, j) - cj

        def hit_body(k, carry):
            hval = hv[pl.ds(n0 + k * LANES, LANES)]
            valid = lanes + k * LANES < cj
            b = lax.bitwise_and(hval, B - 1)
            col = lax.shift_right_logical(hval, 12) - j * BLK
            if j < NBLK - 1:
                for r in range(8):
                    rfull = jnp.full((LANES,), r, jnp.int32)
                    vals = plsc.load_gather(buf, [rfull, col], mask=valid)
                    plsc.store_scatter(out_v, [rfull, b], vals, mask=valid)
            else:
                vs = valid & (col < TSPLIT)
                vt = valid & (col >= TSPLIT)
                tcol = col + 19 * BLK - (V - TW)
                for r in range(8):
                    rfull = jnp.full((LANES,), r, jnp.int32)
                    vals = plsc.load_gather(buf, [rfull, col], mask=vs)
                    plsc.store_scatter(out_v, [rfull, b], vals, mask=vs)
                    tvals = plsc.load_gather(tail_v, [rfull, tcol], mask=vt)
                    plsc.store_scatter(out_v, [rfull, b], tvals, mask=vt)
            return carry

        lax.fori_loop(0, (cj + LANES - 1) // LANES, hit_body, 0)
        if j + 2 < NBLK:
            cps[j + 2] = fire(j + 2)
    pltpu.sync_copy(out_v, out_hbm.at[e, pl.ds(r8, 8), :])


def kernel(indices, weight):
    wt = weight.transpose(0, 2, 1)
    tail = weight[:, V - TW:, :].transpose(0, 2, 1)
    out = _gather(indices.astype(jnp.int32).reshape(-1), wt, tail)
    return out.transpose(0, 2, 1)


# speedup vs baseline: 1.0676x; 1.0323x over previous
import functools

import jax
import jax.numpy as jnp
from jax import lax
from jax.experimental import pallas as pl
from jax.experimental.pallas import tpu as pltpu
from jax.experimental.pallas import tpu_sc as plsc

E = 8
V = 100000
D = 32
B = 4096

NC = 2
NS = 16
LANES = 16

VPAD = 100096
BLK = 5120
NBLK = 20
_BLKW = [BLK] * 19 + [2688]
TW = 128
TSPLIT = (V - TW) - 19 * BLK
NIV = B // LANES

_mesh = plsc.VectorSubcoreMesh(core_axis_name="c", subcore_axis_name="s")


@functools.partial(
    pl.kernel,
    mesh=_mesh,
    out_type=jax.ShapeDtypeStruct((E, D, B), jnp.float32),
    scratch_types=[
        pltpu.VMEM((B,), jnp.int32),
        pltpu.VMEM((B + NBLK * LANES,), jnp.int32),
        pltpu.VMEM((LANES,), jnp.int32),
        pltpu.VMEM((NBLK, LANES), jnp.int32),
        pltpu.VMEM((NBLK, LANES), jnp.int32),
        pltpu.VMEM((2, 8, BLK), jnp.float32),
        pltpu.VMEM((8, TW), jnp.float32),
        pltpu.VMEM((8, B), jnp.float32),
        pltpu.SemaphoreType.DMA,
        pltpu.SemaphoreType.DMA,
    ],
    compiler_params=pltpu.CompilerParams(
        use_tc_tiling_on_sc=True,
        disable_bounds_checks=True,
        needs_layout_passes=False,
    ),
)
def _gather(idx_hbm, wt_hbm, tail_hbm, out_hbm, idx_v, hv, tmp_v, cnt_v,
            offs_v, bufs, tail_v, out_v, sem_a, sem_b):
    wid = lax.axis_index("s") * NC + lax.axis_index("c")
    e = wid // 4
    r8 = pl.multiple_of((wid % 4) * 8, 8)
    lanes = lax.iota(jnp.int32, LANES)
    pltpu.sync_copy(idx_hbm.at[pl.ds(pl.multiple_of(e * B, B), B)], idx_v)
    pltpu.sync_copy(tail_hbm.at[e, pl.ds(r8, 8), :], tail_v)

    def bucket_of(v):
        return lax.shift_right_logical(
            lax.shift_right_logical(v, 10) * 13108, 16
        )

    def fire(j):
        w = _BLKW[j]
        return pltpu.async_copy(
            wt_hbm.at[e, pl.ds(r8, 8), pl.ds(j * BLK, w)],
            bufs.at[j % 2, :, pl.ds(0, w)],
            sem_a if j % 2 == 0 else sem_b,
        )

    cps = {0: fire(0), 1: fire(1)}

    zeros = jnp.zeros((LANES,), jnp.int32)
    for j in range(NBLK):
        cnt_v[j] = zeros

    ones = jnp.full((LANES,), 1, jnp.int32)

    def cnt_body(i2, carry):
        for u in range(2):
            v = idx_v[pl.ds((i2 * 2 + u) * LANES, LANES)]
            plsc.addupdate_scatter(cnt_v, [bucket_of(v), lanes], ones)
        return carry

    lax.fori_loop(0, NIV // 2, cnt_body, 0)

    tot = jnp.int32(0)
    for j in range(NBLK):
        row = cnt_v[j]
        offs_v[j] = plsc.cumsum(row) - row + tot
        tot = tot + lax.bitwise_and(jnp.sum(row) + (LANES - 1), -LANES)

    def place_body(i2, carry):
        for u in range(2):
            i = i2 * 2 + u
            v = idx_v[pl.ds(i * LANES, LANES)]
            bid = bucket_of(v)
            packed = v * B + (lanes + i * LANES)
            dst = plsc.load_gather(offs_v, [bid, lanes])
            plsc.store_scatter(hv, [dst], packed)
            plsc.addupdate_scatter(offs_v, [bid, lanes], ones)
        return carry

    lax.fori_loop(0, NIV // 2, place_body, 0)

    for j in range(NBLK):
        cps[j].wait()
        buf = bufs.at[j % 2]
        crow = cnt_v[j]
        cj = jnp.sum(crow)
        lane0 = lanes == 0
        n0 = jnp.sum(jnp.where(lane0, offs_v[j] - crow, 0))

        def hit_body(k, carry):
            hval = hv[pl.ds(n0 + k * LANES, LANES)]
            valid = lanes + k * LANES < cj
            b = lax.bitwise_and(hval, B - 1)
            col = lax.shift_right_logical(hval, 12) - j * BLK
            if j < NBLK - 1:
                for r in range(8):
                    rfull = jnp.full((LANES,), r, jnp.int32)
                    vals = plsc.load_gather(buf, [rfull, col], mask=valid)
                    plsc.store_scatter(out_v, [rfull, b], vals, mask=valid)
            else:
                vs = valid & (col < TSPLIT)
                vt = valid & (col >= TSPLIT)
                tcol = col + 19 * BLK - (V - TW)
                for r in range(8):
                    rfull = jnp.full((LANES,), r, jnp.int32)
                    vals = plsc.load_gather(buf, [rfull, col], mask=vs)
                    plsc.store_scatter(out_v, [rfull, b], vals, mask=vs)
                    tvals = plsc.load_gather(tail_v, [rfull, tcol], mask=vt)
                    plsc.store_scatter(out_v, [rfull, b], tvals, mask=vt)
            return carry

        lax.fori_loop(0, (cj + LANES - 1) // LANES, hit_body, 0)
        if j + 2 < NBLK:
            cps[j + 2] = fire(j + 2)
    pltpu.sync_copy(out_v, out_hbm.at[e, pl.ds(r8, 8), :])


def kernel(indices, weight):
    wt = weight.transpose(0, 2, 1)
    tail = weight[:, V - TW:, :].transpose(0, 2, 1)
    out = _gather(indices.astype(jnp.int32).reshape(-1), wt, tail)
    return out.transpose(0, 2, 1)


# speedup vs baseline: 1.0708x; 1.0029x over previous
import functools

import jax
import jax.numpy as jnp
from jax import lax
from jax.experimental import pallas as pl
from jax.experimental.pallas import tpu as pltpu
from jax.experimental.pallas import tpu_sc as plsc

E = 8
V = 100000
D = 32
B = 4096

NC = 2
NS = 16
LANES = 16

VPAD = 100096
BLK = 5120
NBLK = 20
_BLKW = [BLK] * 19 + [2688]
TW = 128
TSPLIT = (V - TW) - 19 * BLK
NIV = B // LANES

_mesh = plsc.VectorSubcoreMesh(core_axis_name="c", subcore_axis_name="s")


@functools.partial(
    pl.kernel,
    mesh=_mesh,
    out_type=jax.ShapeDtypeStruct((E, D, B), jnp.float32),
    scratch_types=[
        pltpu.VMEM((B,), jnp.int32),
        pltpu.VMEM((B + NBLK * LANES,), jnp.int32),
        pltpu.VMEM((NBLK, LANES), jnp.int32),
        pltpu.VMEM((NBLK, LANES), jnp.int32),
        pltpu.VMEM((2, 8, BLK), jnp.float32),
        pltpu.VMEM((8, TW), jnp.float32),
        pltpu.VMEM((8, B), jnp.float32),
        pltpu.SemaphoreType.DMA,
        pltpu.SemaphoreType.DMA,
    ],
    compiler_params=pltpu.CompilerParams(
        use_tc_tiling_on_sc=True,
        disable_bounds_checks=True,
        needs_layout_passes=False,
    ),
)
def _gather(idx_hbm, wt_hbm, tail_hbm, out_hbm, idx_v, hv, cnt_v,
            offs_v, bufs, tail_v, out_v, sem_a, sem_b):
    wid = lax.axis_index("s") * NC + lax.axis_index("c")
    e = wid // 4
    r8 = pl.multiple_of((wid % 4) * 8, 8)
    lanes = lax.iota(jnp.int32, LANES)
    pltpu.sync_copy(idx_hbm.at[pl.ds(pl.multiple_of(e * B, B), B)], idx_v)
    pltpu.sync_copy(tail_hbm.at[e, pl.ds(r8, 8), :], tail_v)

    def bucket_of(v):
        return lax.shift_right_logical(
            lax.shift_right_logical(v, 10) * 13108, 16
        )

    def fire(j):
        w = _BLKW[j]
        return pltpu.async_copy(
            wt_hbm.at[e, pl.ds(r8, 8), pl.ds(j * BLK, w)],
            bufs.at[j % 2, :, pl.ds(0, w)],
            sem_a if j % 2 == 0 else sem_b,
        )

    cps = {0: fire(0), 1: fire(1)}

    zeros = jnp.zeros((LANES,), jnp.int32)
    for j in range(NBLK):
        cnt_v[j] = zeros

    ones = jnp.full((LANES,), 1, jnp.int32)

    def cnt_body(i2, carry):
        for u in range(2):
            v = idx_v[pl.ds((i2 * 2 + u) * LANES, LANES)]
            plsc.addupdate_scatter(cnt_v, [bucket_of(v), lanes], ones)
        return carry

    lax.fori_loop(0, NIV // 2, cnt_body, 0)

    tot = jnp.int32(0)
    for j in range(NBLK):
        row = cnt_v[j]
        offs_v[j] = plsc.cumsum(row) - row + tot
        tot = tot + lax.bitwise_and(jnp.sum(row) + (LANES - 1), -LANES)

    def place_body(i2, carry):
        for u in range(2):
            i = i2 * 2 + u
            v = idx_v[pl.ds(i * LANES, LANES)]
            bid = bucket_of(v)
            packed = v * B + (lanes + i * LANES)
            dst = plsc.load_gather(offs_v, [bid, lanes])
            plsc.store_scatter(hv, [dst], packed)
            plsc.addupdate_scatter(offs_v, [bid, lanes], ones)
        return carry

    lax.fori_loop(0, NIV // 2, place_body, 0)

    for j in range(NBLK):
        cps[j].wait()
        buf = bufs.at[j % 2]
        crow = cnt_v[j]
        cj = jnp.sum(crow)
        lane0 = lanes == 0
        n0 = jnp.sum(jnp.where(lane0, offs_v[j] - crow, 0))

        def hit_body(k, carry):
            hval = hv[pl.ds(n0 + k * LANES, LANES)]
            valid = lanes + k * LANES < cj
            b = lax.bitwise_and(hval, B - 1)
            col = lax.shift_right_logical(hval, 12) - j * BLK
            if j < NBLK - 1:
                for r in range(8):
                    rfull = jnp.full((LANES,), r, jnp.int32)
                    vals = plsc.load_gather(buf, [rfull, col], mask=valid)
                    plsc.store_scatter(out_v, [rfull, b], vals, mask=valid)
            else:
                vs = valid & (col < TSPLIT)
                vt = valid & (col >= TSPLIT)
                tcol = col + 19 * BLK - (V - TW)
                for r in range(8):
                    rfull = jnp.full((LANES,), r, jnp.int32)
                    vals = plsc.load_gather(buf, [rfull, col], mask=vs)
                    plsc.store_scatter(out_v, [rfull, b], vals, mask=vs)
                    tvals = plsc.load_gather(tail_v, [rfull, tcol], mask=vt)
                    plsc.store_scatter(out_v, [rfull, b], tvals, mask=vt)
            return carry

        lax.fori_loop(0, (cj + LANES - 1) // LANES, hit_body, 0)
        if j + 2 < NBLK:
            cps[j + 2] = fire(j + 2)
    pltpu.sync_copy(out_v, out_hbm.at[e, pl.ds(r8, 8), :])


def kernel(indices, weight):
    wt = weight.transpose(0, 2, 1)
    tail = weight[:, V - TW:, :].transpose(0, 2, 1)
    out = _gather(indices.astype(jnp.int32).reshape(-1), wt, tail)
    return out.transpose(0, 2, 1)
